# Initial kernel scaffold; baseline (speedup 1.0000x reference)
import functools

import jax
import jax.numpy as jnp
from jax import lax
from jax.experimental import pallas as pl
from jax.experimental.pallas import tpu as pltpu
from jax.experimental.pallas import tpu_sc as plsc

N = 10000
D = 128
NC = 2
NS = 16
NW = NC * NS
CHUNK = 128
LANES = 16
ROWS_PER_TILE = N // NS


@functools.lru_cache(maxsize=None)
def _spmm_call(steps: int):
    mesh = plsc.VectorSubcoreMesh(core_axis_name="c", subcore_axis_name="s")

    @functools.partial(
        pl.kernel,
        out_type=jax.ShapeDtypeStruct((NC * N, D), jnp.float32),
        mesh=mesh,
        scratch_types=[
            pltpu.VMEM((CHUNK,), jnp.int32),
            pltpu.VMEM((CHUNK,), jnp.int32),
            pltpu.VMEM((CHUNK,), jnp.float32),
            pltpu.VMEM((CHUNK, D), jnp.float32),
            pltpu.VMEM_SHARED((N, D), jnp.float32),
        ],
    )
    def spmm(x_hbm, src_hbm, dst_hbm, w_hbm, out_hbm, sidx, didx, wbuf, rows, acc):
        cid = lax.axis_index("c")
        sid = lax.axis_index("s")
        wid = cid * NS + sid

        def zrow(i, carry):
            for j in range(D // LANES):
                rows[i, pl.ds(j * LANES, LANES)] = jnp.zeros((LANES,), jnp.float32)
            return carry

        lax.fori_loop(0, CHUNK, zrow, 0)
        base_r = sid * ROWS_PER_TILE
        for k in range(ROWS_PER_TILE // CHUNK):
            pltpu.sync_copy(rows, acc.at[pl.ds(base_r + k * CHUNK, CHUNK)])
        rem = ROWS_PER_TILE % CHUNK
        if rem:
            pltpu.sync_copy(
                rows.at[pl.ds(0, rem)],
                acc.at[pl.ds(base_r + (ROWS_PER_TILE // CHUNK) * CHUNK, rem)],
            )
        plsc.subcore_barrier()

        ebase = wid * (steps * CHUNK)

        def step(g, carry):
            off = ebase + g * CHUNK
            pltpu.sync_copy(src_hbm.at[pl.ds(off, CHUNK)], sidx)
            pltpu.sync_copy(dst_hbm.at[pl.ds(off, CHUNK)], didx)
            pltpu.sync_copy(w_hbm.at[pl.ds(off, CHUNK)], wbuf)
            pltpu.sync_copy(x_hbm.at[sidx], rows)

            def scale(i, c2):
                wi = wbuf[i]
                for j in range(D // LANES):
                    rows[i, pl.ds(j * LANES, LANES)] = (
                        rows[i, pl.ds(j * LANES, LANES)] * wi
                    )
                return c2

            lax.fori_loop(0, CHUNK, scale, 0)
            pltpu.sync_copy(rows, acc.at[didx], add=True)
            return carry

        lax.fori_loop(0, steps, step, 0)

        plsc.subcore_barrier()
        pltpu.sync_copy(
            acc.at[pl.ds(base_r, ROWS_PER_TILE)],
            out_hbm.at[pl.ds(cid * N + base_r, ROWS_PER_TILE)],
        )

    return spmm


BLK = 400


def _linear_body(x_ref, p0_ref, p1_ref, w1_ref, w2_ref, b_ref, o_ref):
    xnb = p0_ref[...] + p1_ref[...]
    y = jnp.dot(x_ref[...], w1_ref[...], preferred_element_type=jnp.float32)
    y = y + jnp.dot(xnb, w2_ref[...], preferred_element_type=jnp.float32)
    y = y + b_ref[...]
    o_ref[...] = jnp.maximum(y, 0.0)


@functools.lru_cache(maxsize=None)
def _linear_call():
    nb = N // BLK
    return pl.pallas_call(
        _linear_body,
        grid=(nb,),
        in_specs=[
            pl.BlockSpec((BLK, D), lambda i: (i, 0)),
            pl.BlockSpec((BLK, D), lambda i: (i, 0)),
            pl.BlockSpec((BLK, D), lambda i: (i + nb, 0)),
            pl.BlockSpec((D, D), lambda i: (0, 0)),
            pl.BlockSpec((D, D), lambda i: (0, 0)),
            pl.BlockSpec((1, D), lambda i: (0, 0)),
        ],
        out_specs=pl.BlockSpec((BLK, D), lambda i: (i, 0)),
        out_shape=jax.ShapeDtypeStruct((N, D), jnp.float32),
    )


def kernel(x, edge_index, edge_weight, W, b):
    E = edge_index.shape[1]
    steps = -(-E // (NW * CHUNK))
    epad = steps * NW * CHUNK
    pad = epad - E
    src = edge_index[0]
    dst = edge_index[1]
    w = edge_weight
    if pad:
        src = jnp.concatenate([src, jnp.zeros((pad,), jnp.int32)])
        dst = jnp.concatenate([dst, jnp.zeros((pad,), jnp.int32)])
        w = jnp.concatenate([w, jnp.zeros((pad,), jnp.float32)])

    part = _spmm_call(steps)(x, src, dst, w)
    return _linear_call()(x, part, part, W[:D], W[D:], b.reshape(1, D))


# speedup vs baseline: 3.3274x; 3.3274x over previous
import functools

import jax
import jax.numpy as jnp
from jax import lax
from jax.experimental import pallas as pl
from jax.experimental.pallas import tpu as pltpu
from jax.experimental.pallas import tpu_sc as plsc

N = 10000
D = 128
NC = 2
NS = 16
NW = NC * NS
CHUNK = 128
LANES = 16
NACC = 12800
ROWS_PER_TILE = NACC // NS


@functools.lru_cache(maxsize=None)
def _spmm_call(steps: int):
    mesh = plsc.VectorSubcoreMesh(core_axis_name="c", subcore_axis_name="s")

    @functools.partial(
        pl.kernel,
        out_type=jax.ShapeDtypeStruct((NC * NACC, D), jnp.float32),
        mesh=mesh,
        scratch_types=[
            pltpu.VMEM((CHUNK,), jnp.int32),
            pltpu.VMEM((CHUNK,), jnp.int32),
            pltpu.VMEM((CHUNK,), jnp.float32),
            pltpu.VMEM((CHUNK, D), jnp.float32),
            pltpu.VMEM_SHARED((NACC, D), jnp.float32),
        ],
    )
    def spmm(x_hbm, src_hbm, dst_hbm, w_hbm, out_hbm, sidx, didx, wbuf, rows, acc):
        cid = lax.axis_index("c")
        sid = lax.axis_index("s")
        wid = cid * NS + sid

        def zrow(i, carry):
            for j in range(D // LANES):
                rows[i, pl.ds(j * LANES, LANES)] = jnp.zeros((LANES,), jnp.float32)
            return carry

        lax.fori_loop(0, CHUNK, zrow, 0)
        base_r = sid * ROWS_PER_TILE
        for k in range(ROWS_PER_TILE // CHUNK):
            pltpu.sync_copy(rows, acc.at[pl.ds(base_r + k * CHUNK, CHUNK)])
        rem = ROWS_PER_TILE % CHUNK
        if rem:
            pltpu.sync_copy(
                rows.at[pl.ds(0, rem)],
                acc.at[pl.ds(base_r + (ROWS_PER_TILE // CHUNK) * CHUNK, rem)],
            )
        plsc.subcore_barrier()

        ebase = wid * (steps * CHUNK)

        def step(g, carry):
            off = ebase + g * CHUNK
            pltpu.sync_copy(src_hbm.at[pl.ds(off, CHUNK)], sidx)
            pltpu.sync_copy(dst_hbm.at[pl.ds(off, CHUNK)], didx)
            pltpu.sync_copy(w_hbm.at[pl.ds(off, CHUNK)], wbuf)
            pltpu.sync_copy(x_hbm.at[sidx], rows)

            def scale(g, c2):
                wv = wbuf[pl.ds(g * LANES, LANES)]
                for lane in range(LANES):
                    wi = wv[lane]
                    r = g * LANES + lane
                    for j in range(D // LANES):
                        rows[r, pl.ds(j * LANES, LANES)] = (
                            rows[r, pl.ds(j * LANES, LANES)] * wi
                        )
                return c2

            lax.fori_loop(0, CHUNK // LANES, scale, 0)
            pltpu.sync_copy(rows, acc.at[didx], add=True)
            return carry

        lax.fori_loop(0, steps, step, 0)

        plsc.subcore_barrier()
        pltpu.sync_copy(
            acc.at[pl.ds(base_r, ROWS_PER_TILE)],
            out_hbm.at[pl.ds(cid * NACC + base_r, ROWS_PER_TILE)],
        )

    return spmm


BLK = 400


def _linear_body(x_ref, p0_ref, p1_ref, w1_ref, w2_ref, b_ref, o_ref):
    xnb = p0_ref[...] + p1_ref[...]
    y = jnp.dot(x_ref[...], w1_ref[...], preferred_element_type=jnp.float32)
    y = y + jnp.dot(xnb, w2_ref[...], preferred_element_type=jnp.float32)
    y = y + b_ref[...]
    o_ref[...] = jnp.maximum(y, 0.0)


@functools.lru_cache(maxsize=None)
def _linear_call():
    nb = N // BLK
    return pl.pallas_call(
        _linear_body,
        grid=(nb,),
        in_specs=[
            pl.BlockSpec((BLK, D), lambda i: (i, 0)),
            pl.BlockSpec((BLK, D), lambda i: (i, 0)),
            pl.BlockSpec((BLK, D), lambda i: (i + NACC // BLK, 0)),
            pl.BlockSpec((D, D), lambda i: (0, 0)),
            pl.BlockSpec((D, D), lambda i: (0, 0)),
            pl.BlockSpec((1, D), lambda i: (0, 0)),
        ],
        out_specs=pl.BlockSpec((BLK, D), lambda i: (i, 0)),
        out_shape=jax.ShapeDtypeStruct((N, D), jnp.float32),
    )


def kernel(x, edge_index, edge_weight, W, b):
    E = edge_index.shape[1]
    steps = -(-E // (NW * CHUNK))
    epad = steps * NW * CHUNK
    pad = epad - E
    src = edge_index[0]
    dst = edge_index[1]
    w = edge_weight
    if pad:
        src = jnp.concatenate([src, jnp.zeros((pad,), jnp.int32)])
        dst = jnp.concatenate([dst, jnp.zeros((pad,), jnp.int32)])
        w = jnp.concatenate([w, jnp.zeros((pad,), jnp.float32)])

    part = _spmm_call(steps)(x, src, dst, w)
    return _linear_call()(x, part, part, W[:D], W[D:], b.reshape(1, D))
